# baseline (speedup 1.0000x reference)
import functools

import jax
import jax.numpy as jnp
from jax import lax
from jax.experimental import pallas as pl
from jax.experimental.pallas import tpu as pltpu
from jax.experimental.pallas import tpu_sc as plsc

N = 10000
D_IN = 128
F1, F2, F3 = 128, 64, 32
K = 16

NPAD = 10240
E = 320000
NC, NS, LANES = 2, 16, 16
CH = 128
CPT = 160
PHASES = 4
CPP = CPT // PHASES
EPAD = CH * CPT * NS
ROWS_PER_TILE = NPAD // NS
DEG_W = 16


def _make_edge_scatter(F, ch, phases, interpret=False):
    mesh = plsc.VectorSubcoreMesh(core_axis_name="c", subcore_axis_name="s",
                                  num_cores=NC, num_subcores=NS)
    cpt = (EPAD // NS) // ch
    cpp = cpt // phases
    n_groups = cpp // 4
    assert cpp % 4 == 0 and ROWS_PER_TILE % ch == 0

    @functools.partial(
        pl.kernel,
        out_type=jax.ShapeDtypeStruct((NC, NPAD, F), jnp.float32),
        mesh=mesh,
        scratch_types=[
            pltpu.VMEM_SHARED((NPAD, F), jnp.float32),
            pltpu.VMEM((cpp, ch), jnp.int32),
            pltpu.VMEM((cpp, ch), jnp.int32),
        ] + [pltpu.VMEM((ch, F), jnp.float32)] * 4
          + [pltpu.SemaphoreType.DMA] * 8,
        compiler_params=pltpu.CompilerParams(use_tc_tiling_on_sc=False),
        interpret=interpret,
    )
    def scatter(z_hbm, src_hbm, dst_hbm, out_hbm, acc_sh, sidx, didx,
                r0, r1, r2, r3, g0, g1, g2, g3, s0, s1, s2, s3):
        c = lax.axis_index("c")
        s = lax.axis_index("s")
        rows = (r0, r1, r2, r3)
        gs = (g0, g1, g2, g3)
        ss = (s0, s1, s2, s3)
        zeros16 = jnp.zeros((LANES,), jnp.float32)

        def zrow(i, carry):
            for j in range(F // LANES):
                r0[i, pl.ds(j * LANES, LANES)] = zeros16
            return carry

        lax.fori_loop(0, ch, zrow, 0)
        for t in range(ROWS_PER_TILE // ch):
            pltpu.sync_copy(
                r0, acc_sh.at[pl.ds(s * ROWS_PER_TILE + t * ch, ch)])
        plsc.subcore_barrier()

        base = (c * NS + s) * cpt
        for h in range(phases):
            pltpu.sync_copy(src_hbm.at[pl.ds(base + h * cpp, cpp)], sidx)
            pltpu.sync_copy(dst_hbm.at[pl.ds(base + h * cpp, cpp)], didx)
            pltpu.async_copy(z_hbm.at[sidx.at[0]], r0, g0)
            pltpu.async_copy(z_hbm.at[sidx.at[1]], r1, g1)

            def group(i, carry):
                for b in range(4):
                    g = 4 * i + b
                    bn = (b + 2) % 4
                    pltpu.make_async_copy(z_hbm.at[sidx.at[g]], rows[b],
                                          gs[b]).wait()
                    pltpu.async_copy(rows[b], acc_sh.at[didx.at[g]],
                                     ss[b], add=True)
                    if b < 2:
                        @pl.when(i > 0)
                        def _():
                            pltpu.make_async_copy(
                                rows[bn], acc_sh.at[didx.at[g - 2]],
                                ss[bn]).wait()
                        pltpu.async_copy(z_hbm.at[sidx.at[g + 2]], rows[bn],
                                         gs[bn])
                    else:
                        pltpu.make_async_copy(
                            rows[bn], acc_sh.at[didx.at[g - 2]],
                            ss[bn]).wait()

                        @pl.when(i < n_groups - 1)
                        def _():
                            pltpu.async_copy(z_hbm.at[sidx.at[g + 2]],
                                             rows[bn], gs[bn])
                return carry

            lax.fori_loop(0, n_groups, group, 0)
            pltpu.make_async_copy(r2, acc_sh.at[didx.at[cpp - 2]],
                                  s2).wait()
            pltpu.make_async_copy(r3, acc_sh.at[didx.at[cpp - 1]],
                                  s3).wait()
        plsc.subcore_barrier()
        pltpu.sync_copy(
            acc_sh.at[pl.ds(s * ROWS_PER_TILE, ROWS_PER_TILE)],
            out_hbm.at[c, pl.ds(s * ROWS_PER_TILE, ROWS_PER_TILE)])

    return scatter


def _make_deg_scatter(interpret=False):
    mesh = plsc.VectorSubcoreMesh(core_axis_name="c", subcore_axis_name="s",
                                  num_cores=NC, num_subcores=NS)

    @functools.partial(
        pl.kernel,
        out_type=jax.ShapeDtypeStruct((NC, NPAD, DEG_W), jnp.float32),
        mesh=mesh,
        scratch_types=[
            pltpu.VMEM_SHARED((NPAD, DEG_W), jnp.float32),
            pltpu.VMEM((CPT, CH), jnp.int32),
            pltpu.VMEM((CH, DEG_W), jnp.float32),
            pltpu.VMEM((CH, DEG_W), jnp.float32),
        ],
        compiler_params=pltpu.CompilerParams(use_tc_tiling_on_sc=False),
        interpret=interpret,
    )
    def deg(dst_hbm, out_hbm, acc_sh, didx, ones, zbuf):
        c = lax.axis_index("c")
        s = lax.axis_index("s")
        zeros16 = jnp.zeros((LANES,), jnp.float32)
        ones16 = jnp.ones((LANES,), jnp.float32)

        def fill(i, carry):
            zbuf[i, pl.ds(0, LANES)] = zeros16
            ones[i, pl.ds(0, LANES)] = ones16
            return carry

        lax.fori_loop(0, CH, fill, 0)
        for t in range(ROWS_PER_TILE // CH):
            pltpu.sync_copy(
                zbuf, acc_sh.at[pl.ds(s * ROWS_PER_TILE + t * CH, CH)])
        base = (c * NS + s) * CPT
        pltpu.sync_copy(dst_hbm.at[pl.ds(base, CPT)], didx)
        plsc.subcore_barrier()

        def chunk(k, carry):
            pltpu.sync_copy(ones, acc_sh.at[didx.at[k]], add=True)
            return carry

        lax.fori_loop(0, CPT, chunk, 0)
        plsc.subcore_barrier()
        pltpu.sync_copy(
            acc_sh.at[pl.ds(s * ROWS_PER_TILE, ROWS_PER_TILE)],
            out_hbm.at[c, pl.ds(s * ROWS_PER_TILE, ROWS_PER_TILE)])

    return deg


def _prep_body(pdeg_ref, x1_ref, x2_ref, w_ref, z_ref, dinv_ref):
    for g, x_ref in ((0, x1_ref), (1, x2_ref)):
        deg = pdeg_ref[g, :, 0:1] + 1.0
        dinv = lax.rsqrt(deg)
        dinv_ref[g * NPAD:(g + 1) * NPAD] = dinv
        h = jnp.dot(x_ref[...], w_ref[...],
                    preferred_element_type=jnp.float32)
        z_ref[g * NPAD:g * NPAD + N, :] = dinv[0:N] * h
        z_ref[g * NPAD + N:(g + 1) * NPAD, :] = jnp.zeros(
            (NPAD - N, F1), jnp.float32)


def _make_prep(interpret=False):
    return pl.pallas_call(
        _prep_body,
        out_shape=[
            jax.ShapeDtypeStruct((2 * NPAD, F1), jnp.float32),
            jax.ShapeDtypeStruct((2 * NPAD, 1), jnp.float32),
        ],
        interpret=interpret,
    )


def _mid_body(p_ref, z_ref, dinv_ref, b_ref, w_ref, out_ref, *, F_out):
    a = p_ref[...] + z_ref[...]
    u = jnp.maximum(dinv_ref[...] * a + b_ref[...], 0.0)
    h = jnp.dot(u, w_ref[...], preferred_element_type=jnp.float32)
    out_ref[...] = dinv_ref[...] * h
    for g in range(2):
        out_ref[g * NPAD + N:(g + 1) * NPAD, :] = jnp.zeros(
            (NPAD - N, F_out), jnp.float32)


def _make_mid(F_out, interpret=False):
    return pl.pallas_call(
        functools.partial(_mid_body, F_out=F_out),
        out_shape=jax.ShapeDtypeStruct((2 * NPAD, F_out), jnp.float32),
        interpret=interpret,
    )


def _final_body(p_ref, z_ref, dinv_ref,
                b3_ref, watt_ref, wt_ref, v_ref, bt_ref,
                fc1_ref, fb1_ref, fc2_ref, fb2_ref, fc3_ref, fb3_ref,
                ws_ref, bs_ref, out_ref):
    def pooled(g):
        sl = pl.ds(g * NPAD, N)
        a = p_ref[sl, :] + z_ref[sl, :]
        u = dinv_ref[sl, :] * a + b3_ref[...]
        m = jnp.dot(u, watt_ref[...], preferred_element_type=jnp.float32)
        ctx = jnp.tanh(jnp.mean(m, axis=0, keepdims=True))
        gate = jax.nn.sigmoid(lax.dot_general(
            u, ctx, (((1,), (1,)), ((), ())),
            preferred_element_type=jnp.float32))
        return lax.dot_general(
            u, gate, (((0,), (0,)), ((), ())),
            preferred_element_type=jnp.float32)

    h1 = pooled(0)
    h2 = pooled(1)

    sks = []
    for k in range(K):
        tk = jnp.dot(wt_ref[k], h2, preferred_element_type=jnp.float32)
        sks.append(lax.dot_general(
            h1, tk, (((0,), (0,)), ((), ())),
            preferred_element_type=jnp.float32))
    scoring = jnp.concatenate(sks, axis=1)
    comb = jnp.concatenate([h1, h2], axis=0)
    block = lax.dot_general(
        comb, v_ref[...], (((0,), (1,)), ((), ())),
        preferred_element_type=jnp.float32)
    srow = jnp.maximum(scoring + block + bt_ref[...], 0.0)
    srow = jnp.maximum(
        jnp.dot(srow, fc1_ref[...], preferred_element_type=jnp.float32)
        + fb1_ref[...], 0.0)
    srow = jnp.maximum(
        jnp.dot(srow, fc2_ref[...], preferred_element_type=jnp.float32)
        + fb2_ref[...], 0.0)
    srow = jnp.maximum(
        jnp.dot(srow, fc3_ref[...], preferred_element_type=jnp.float32)
        + fb3_ref[...], 0.0)
    out_ref[...] = jax.nn.sigmoid(
        jnp.dot(srow, ws_ref[...], preferred_element_type=jnp.float32)
        + bs_ref[...])


def _make_final(interpret=False):
    return pl.pallas_call(
        _final_body,
        out_shape=jax.ShapeDtypeStruct((1, 1), jnp.float32),
        interpret=interpret,
    )


@functools.lru_cache(maxsize=None)
def _kernels():
    return dict(
        deg=_make_deg_scatter(),
        scat128=_make_edge_scatter(F1, 64, 8),
        scat64=_make_edge_scatter(F2, 128, 1),
        scat32=_make_edge_scatter(F3, 128, 1),
        prep=_make_prep(),
        mid64=_make_mid(F2),
        mid32=_make_mid(F3),
        final=_make_final(),
    )


def _pad_edges(ei1, ei2):
    fill1 = jnp.full((EPAD - E,), N, dtype=jnp.int32)
    fill2 = jnp.full((EPAD - E,), NPAD + N, dtype=jnp.int32)
    src = jnp.concatenate([ei1[0], fill1, ei2[0] + NPAD, fill2])
    dst = jnp.concatenate([ei1[1], fill1, ei2[1], fill1])
    return src, dst


def kernel(x1, edge_index_1, x2, edge_index_2, W1, b1, W2, b2, W3, b3,
           Watt, Wt, V, bt, FC1, fb1, FC2, fb2, FC3, fb3, WS, bs):
    ks = _kernels()
    src, dst = _pad_edges(edge_index_1, edge_index_2)
    src64, dst64 = src.reshape(-1, 64), dst.reshape(-1, 64)
    src128, dst128 = src.reshape(-1, CH), dst.reshape(-1, CH)
    pdeg = ks["deg"](dst128)
    z1, dinv = ks["prep"](pdeg, x1, x2, W1)
    p1 = ks["scat128"](z1, src64, dst64).reshape(2 * NPAD, F1)
    z2 = ks["mid64"](p1, z1, dinv, b1.reshape(1, F1), W2)
    p2 = ks["scat64"](z2, src128, dst128).reshape(2 * NPAD, F2)
    z3 = ks["mid32"](p2, z2, dinv, b2.reshape(1, F2), W3)
    p3 = ks["scat32"](z3, src128, dst128).reshape(2 * NPAD, F3)
    wt_r = jnp.transpose(Wt, (2, 0, 1))
    score = ks["final"](p3, z3, dinv,
                        b3.reshape(1, F3), Watt, wt_r, V, bt.reshape(1, K),
                        FC1, fb1.reshape(1, -1), FC2, fb2.reshape(1, -1),
                        FC3, fb3.reshape(1, -1), WS, bs.reshape(1, 1))
    return score.reshape(-1)


# speedup vs baseline: 1.5356x; 1.5356x over previous
import functools

import jax
import jax.numpy as jnp
from jax import lax
from jax.experimental import pallas as pl
from jax.experimental.pallas import tpu as pltpu
from jax.experimental.pallas import tpu_sc as plsc

N = 10000
D_IN = 128
F1, F2, F3 = 128, 64, 32
K = 16

NPAD = 10240
E = 320000
NC, NS, LANES = 2, 16, 16
CH = 128
CPT = 160
PHASES = 4
CPP = CPT // PHASES
EPAD = CH * CPT * NS
ROWS_PER_TILE = NPAD // NS
DEG_W = 16


def _make_edge_scatter(F, ch, phases, interpret=False):
    mesh = plsc.VectorSubcoreMesh(core_axis_name="c", subcore_axis_name="s",
                                  num_cores=NC, num_subcores=NS)
    cpt = (EPAD // NS) // ch
    cpp = cpt // phases
    n_groups = cpp // 4
    assert cpp % 4 == 0 and ROWS_PER_TILE % ch == 0

    @functools.partial(
        pl.kernel,
        out_type=jax.ShapeDtypeStruct((NC, NPAD, F), jnp.bfloat16),
        mesh=mesh,
        scratch_types=[
            pltpu.VMEM_SHARED((NPAD, F), jnp.bfloat16),
            pltpu.VMEM((cpp, ch), jnp.int32),
            pltpu.VMEM((cpp, ch), jnp.int32),
        ] + [pltpu.VMEM((ch, F), jnp.bfloat16)] * 4
          + [pltpu.SemaphoreType.DMA] * 8,
        compiler_params=pltpu.CompilerParams(use_tc_tiling_on_sc=False),
        interpret=interpret,
    )
    def scatter(z_hbm, src_hbm, dst_hbm, out_hbm, acc_sh, sidx, didx,
                r0, r1, r2, r3, g0, g1, g2, g3, s0, s1, s2, s3):
        c = lax.axis_index("c")
        s = lax.axis_index("s")
        rows = (r0, r1, r2, r3)
        gs = (g0, g1, g2, g3)
        ss = (s0, s1, s2, s3)
        zeros32 = jnp.zeros((2 * LANES,), jnp.bfloat16)

        def zrow(i, carry):
            for j in range(F // (2 * LANES)):
                r0[i, pl.ds(j * 2 * LANES, 2 * LANES)] = zeros32
            return carry

        lax.fori_loop(0, ch, zrow, 0)
        for t in range(ROWS_PER_TILE // ch):
            pltpu.sync_copy(
                r0, acc_sh.at[pl.ds(s * ROWS_PER_TILE + t * ch, ch)])
        plsc.subcore_barrier()

        base = (c * NS + s) * cpt
        for h in range(phases):
            pltpu.sync_copy(src_hbm.at[pl.ds(base + h * cpp, cpp)], sidx)
            pltpu.sync_copy(dst_hbm.at[pl.ds(base + h * cpp, cpp)], didx)
            pltpu.async_copy(z_hbm.at[sidx.at[0]], r0, g0)
            pltpu.async_copy(z_hbm.at[sidx.at[1]], r1, g1)

            def group(i, carry):
                for b in range(4):
                    g = 4 * i + b
                    bn = (b + 2) % 4
                    pltpu.make_async_copy(z_hbm.at[sidx.at[g]], rows[b],
                                          gs[b]).wait()
                    pltpu.async_copy(rows[b], acc_sh.at[didx.at[g]],
                                     ss[b], add=True)
                    if b < 2:
                        @pl.when(i > 0)
                        def _():
                            pltpu.make_async_copy(
                                rows[bn], acc_sh.at[didx.at[g - 2]],
                                ss[bn]).wait()
                        pltpu.async_copy(z_hbm.at[sidx.at[g + 2]], rows[bn],
                                         gs[bn])
                    else:
                        pltpu.make_async_copy(
                            rows[bn], acc_sh.at[didx.at[g - 2]],
                            ss[bn]).wait()

                        @pl.when(i < n_groups - 1)
                        def _():
                            pltpu.async_copy(z_hbm.at[sidx.at[g + 2]],
                                             rows[bn], gs[bn])
                return carry

            lax.fori_loop(0, n_groups, group, 0)
            pltpu.make_async_copy(r2, acc_sh.at[didx.at[cpp - 2]],
                                  s2).wait()
            pltpu.make_async_copy(r3, acc_sh.at[didx.at[cpp - 1]],
                                  s3).wait()
        plsc.subcore_barrier()
        pltpu.sync_copy(
            acc_sh.at[pl.ds(s * ROWS_PER_TILE, ROWS_PER_TILE)],
            out_hbm.at[c, pl.ds(s * ROWS_PER_TILE, ROWS_PER_TILE)])

    return scatter


def _make_deg_scatter(interpret=False):
    mesh = plsc.VectorSubcoreMesh(core_axis_name="c", subcore_axis_name="s",
                                  num_cores=NC, num_subcores=NS)

    @functools.partial(
        pl.kernel,
        out_type=jax.ShapeDtypeStruct((NC, NPAD, DEG_W), jnp.float32),
        mesh=mesh,
        scratch_types=[
            pltpu.VMEM_SHARED((NPAD, DEG_W), jnp.float32),
            pltpu.VMEM((CPT, CH), jnp.int32),
            pltpu.VMEM((CH, DEG_W), jnp.float32),
            pltpu.VMEM((CH, DEG_W), jnp.float32),
        ],
        compiler_params=pltpu.CompilerParams(use_tc_tiling_on_sc=False),
        interpret=interpret,
    )
    def deg(dst_hbm, out_hbm, acc_sh, didx, ones, zbuf):
        c = lax.axis_index("c")
        s = lax.axis_index("s")
        zeros16 = jnp.zeros((LANES,), jnp.float32)
        ones16 = jnp.ones((LANES,), jnp.float32)

        def fill(i, carry):
            zbuf[i, pl.ds(0, LANES)] = zeros16
            ones[i, pl.ds(0, LANES)] = ones16
            return carry

        lax.fori_loop(0, CH, fill, 0)
        for t in range(ROWS_PER_TILE // CH):
            pltpu.sync_copy(
                zbuf, acc_sh.at[pl.ds(s * ROWS_PER_TILE + t * CH, CH)])
        base = (c * NS + s) * CPT
        pltpu.sync_copy(dst_hbm.at[pl.ds(base, CPT)], didx)
        plsc.subcore_barrier()

        def chunk(k, carry):
            pltpu.sync_copy(ones, acc_sh.at[didx.at[k]], add=True)
            return carry

        lax.fori_loop(0, CPT, chunk, 0)
        plsc.subcore_barrier()
        pltpu.sync_copy(
            acc_sh.at[pl.ds(s * ROWS_PER_TILE, ROWS_PER_TILE)],
            out_hbm.at[c, pl.ds(s * ROWS_PER_TILE, ROWS_PER_TILE)])

    return deg


def _prep_body(pdeg_ref, x1_ref, x2_ref, w_ref, z_ref, dinv_ref):
    for g, x_ref in ((0, x1_ref), (1, x2_ref)):
        deg = pdeg_ref[g, :, 0:1] + 1.0
        dinv = lax.rsqrt(deg)
        dinv_ref[g * NPAD:(g + 1) * NPAD] = dinv
        h = jnp.dot(x_ref[...], w_ref[...],
                    preferred_element_type=jnp.float32)
        z_ref[g * NPAD:g * NPAD + N, :] = (dinv[0:N] * h).astype(
            jnp.bfloat16)
        z_ref[g * NPAD + N:(g + 1) * NPAD, :] = jnp.zeros(
            (NPAD - N, F1), jnp.bfloat16)


def _make_prep(interpret=False):
    return pl.pallas_call(
        _prep_body,
        out_shape=[
            jax.ShapeDtypeStruct((2 * NPAD, F1), jnp.bfloat16),
            jax.ShapeDtypeStruct((2 * NPAD, 1), jnp.float32),
        ],
        interpret=interpret,
    )


def _mid_body(p_ref, z_ref, dinv_ref, b_ref, w_ref, out_ref, *, F_out):
    a = (p_ref[...].astype(jnp.float32) + z_ref[...].astype(jnp.float32))
    u = jnp.maximum(dinv_ref[...] * a + b_ref[...], 0.0)
    h = jnp.dot(u, w_ref[...], preferred_element_type=jnp.float32)
    out_ref[...] = (dinv_ref[...] * h).astype(jnp.bfloat16)
    for g in range(2):
        out_ref[g * NPAD + N:(g + 1) * NPAD, :] = jnp.zeros(
            (NPAD - N, F_out), jnp.bfloat16)


def _make_mid(F_out, interpret=False):
    return pl.pallas_call(
        functools.partial(_mid_body, F_out=F_out),
        out_shape=jax.ShapeDtypeStruct((2 * NPAD, F_out), jnp.bfloat16),
        interpret=interpret,
    )


def _final_body(p_ref, z_ref, dinv_ref,
                b3_ref, watt_ref, wt_ref, v_ref, bt_ref,
                fc1_ref, fb1_ref, fc2_ref, fb2_ref, fc3_ref, fb3_ref,
                ws_ref, bs_ref, out_ref):
    def pooled(g):
        sl = pl.ds(g * NPAD, N)
        a = (p_ref[sl, :].astype(jnp.float32)
             + z_ref[sl, :].astype(jnp.float32))
        u = dinv_ref[sl, :] * a + b3_ref[...]
        m = jnp.dot(u, watt_ref[...], preferred_element_type=jnp.float32)
        ctx = jnp.tanh(jnp.mean(m, axis=0, keepdims=True))
        gate = jax.nn.sigmoid(lax.dot_general(
            u, ctx, (((1,), (1,)), ((), ())),
            preferred_element_type=jnp.float32))
        return lax.dot_general(
            u, gate, (((0,), (0,)), ((), ())),
            preferred_element_type=jnp.float32)

    h1 = pooled(0)
    h2 = pooled(1)

    sks = []
    for k in range(K):
        tk = jnp.dot(wt_ref[k], h2, preferred_element_type=jnp.float32)
        sks.append(lax.dot_general(
            h1, tk, (((0,), (0,)), ((), ())),
            preferred_element_type=jnp.float32))
    scoring = jnp.concatenate(sks, axis=1)
    comb = jnp.concatenate([h1, h2], axis=0)
    block = lax.dot_general(
        comb, v_ref[...], (((0,), (1,)), ((), ())),
        preferred_element_type=jnp.float32)
    srow = jnp.maximum(scoring + block + bt_ref[...], 0.0)
    srow = jnp.maximum(
        jnp.dot(srow, fc1_ref[...], preferred_element_type=jnp.float32)
        + fb1_ref[...], 0.0)
    srow = jnp.maximum(
        jnp.dot(srow, fc2_ref[...], preferred_element_type=jnp.float32)
        + fb2_ref[...], 0.0)
    srow = jnp.maximum(
        jnp.dot(srow, fc3_ref[...], preferred_element_type=jnp.float32)
        + fb3_ref[...], 0.0)
    out_ref[...] = jax.nn.sigmoid(
        jnp.dot(srow, ws_ref[...], preferred_element_type=jnp.float32)
        + bs_ref[...])


def _make_final(interpret=False):
    return pl.pallas_call(
        _final_body,
        out_shape=jax.ShapeDtypeStruct((1, 1), jnp.float32),
        interpret=interpret,
    )


@functools.lru_cache(maxsize=None)
def _kernels():
    return dict(
        deg=_make_deg_scatter(),
        scat128=_make_edge_scatter(F1, 64, 8),
        scat64=_make_edge_scatter(F2, 128, 1),
        scat32=_make_edge_scatter(F3, 128, 1),
        prep=_make_prep(),
        mid64=_make_mid(F2),
        mid32=_make_mid(F3),
        final=_make_final(),
    )


def _pad_edges(ei1, ei2):
    fill1 = jnp.full((EPAD - E,), N, dtype=jnp.int32)
    fill2 = jnp.full((EPAD - E,), NPAD + N, dtype=jnp.int32)
    src = jnp.concatenate([ei1[0], fill1, ei2[0] + NPAD, fill2])
    dst = jnp.concatenate([ei1[1], fill1, ei2[1], fill1])
    return src, dst


def kernel(x1, edge_index_1, x2, edge_index_2, W1, b1, W2, b2, W3, b3,
           Watt, Wt, V, bt, FC1, fb1, FC2, fb2, FC3, fb3, WS, bs):
    ks = _kernels()
    src, dst = _pad_edges(edge_index_1, edge_index_2)
    src64, dst64 = src.reshape(-1, 64), dst.reshape(-1, 64)
    src128, dst128 = src.reshape(-1, CH), dst.reshape(-1, CH)
    pdeg = ks["deg"](dst128)
    z1, dinv = ks["prep"](pdeg, x1, x2, W1)
    p1 = ks["scat128"](z1, src64, dst64).reshape(2 * NPAD, F1)
    z2 = ks["mid64"](p1, z1, dinv, b1.reshape(1, F1), W2)
    p2 = ks["scat64"](z2, src128, dst128).reshape(2 * NPAD, F2)
    z3 = ks["mid32"](p2, z2, dinv, b2.reshape(1, F2), W3)
    p3 = ks["scat32"](z3, src128, dst128).reshape(2 * NPAD, F3)
    wt_r = jnp.transpose(Wt, (2, 0, 1))
    score = ks["final"](p3, z3, dinv,
                        b3.reshape(1, F3), Watt, wt_r, V, bt.reshape(1, K),
                        FC1, fb1.reshape(1, -1), FC2, fb2.reshape(1, -1),
                        FC3, fb3.reshape(1, -1), WS, bs.reshape(1, 1))
    return score.reshape(-1)


# speedup vs baseline: 2.6626x; 1.7339x over previous
import functools

import jax
import jax.numpy as jnp
from jax import lax
from jax.experimental import pallas as pl
from jax.experimental.pallas import tpu as pltpu
from jax.experimental.pallas import tpu_sc as plsc

N = 10000
D_IN = 128
F1, F2, F3 = 128, 64, 32
K = 16

NPAD = 10240
E = 320000
NC, NS, LANES = 2, 16, 16
CH = 128
CPT = 160
PHASES = 4
CPP = CPT // PHASES
EPAD = CH * CPT * NS
ROWS_PER_TILE = NPAD // NS
DEG_W = 16


def _make_edge_scatter(F, ch, phases, interpret=False):
    mesh = plsc.VectorSubcoreMesh(core_axis_name="c", subcore_axis_name="s",
                                  num_cores=NC, num_subcores=NS)
    cpt = (EPAD // NS) // ch
    cpp = cpt // phases
    n_groups = cpp // 4
    assert cpp % 4 == 0 and ROWS_PER_TILE % ch == 0

    @functools.partial(
        pl.kernel,
        out_type=jax.ShapeDtypeStruct((NC, NPAD, F), jnp.bfloat16),
        mesh=mesh,
        scratch_types=[
            pltpu.VMEM_SHARED((NPAD, F), jnp.bfloat16),
            pltpu.VMEM_SHARED((NPAD, F), jnp.bfloat16),
            pltpu.VMEM((cpp, ch), jnp.int32),
            pltpu.VMEM((cpp, ch), jnp.int32),
        ] + [pltpu.VMEM((ch, F), jnp.bfloat16)] * 4
          + [pltpu.SemaphoreType.DMA] * 8,
        compiler_params=pltpu.CompilerParams(use_tc_tiling_on_sc=False),
        interpret=interpret,
    )
    def scatter(z_hbm, src_hbm, dst_hbm, out_hbm, acc_sh, tab_sh, sidx, didx,
                r0, r1, r2, r3, g0, g1, g2, g3, s0, s1, s2, s3):
        c = lax.axis_index("c")
        s = lax.axis_index("s")
        rows = (r0, r1, r2, r3)
        gs = (g0, g1, g2, g3)
        ss = (s0, s1, s2, s3)
        zeros32 = jnp.zeros((2 * LANES,), jnp.bfloat16)

        def zrow(i, carry):
            for j in range(F // (2 * LANES)):
                r0[i, pl.ds(j * 2 * LANES, 2 * LANES)] = zeros32
            return carry

        lax.fori_loop(0, ch, zrow, 0)
        for t in range(ROWS_PER_TILE // ch):
            pltpu.sync_copy(
                r0, acc_sh.at[pl.ds(s * ROWS_PER_TILE + t * ch, ch)])
        pltpu.sync_copy(
            z_hbm.at[pl.ds(c * NPAD + s * ROWS_PER_TILE, ROWS_PER_TILE)],
            tab_sh.at[pl.ds(s * ROWS_PER_TILE, ROWS_PER_TILE)])
        plsc.subcore_barrier()

        base = (c * NS + s) * cpt
        for h in range(phases):
            pltpu.sync_copy(src_hbm.at[pl.ds(base + h * cpp, cpp)], sidx)
            pltpu.sync_copy(dst_hbm.at[pl.ds(base + h * cpp, cpp)], didx)
            pltpu.async_copy(tab_sh.at[sidx.at[0]], r0, g0)
            pltpu.async_copy(tab_sh.at[sidx.at[1]], r1, g1)

            def group(i, carry):
                for b in range(4):
                    g = 4 * i + b
                    bn = (b + 2) % 4
                    pltpu.make_async_copy(tab_sh.at[sidx.at[g]], rows[b],
                                          gs[b]).wait()
                    pltpu.async_copy(rows[b], acc_sh.at[didx.at[g]],
                                     ss[b], add=True)
                    if b < 2:
                        @pl.when(i > 0)
                        def _():
                            pltpu.make_async_copy(
                                rows[bn], acc_sh.at[didx.at[g - 2]],
                                ss[bn]).wait()
                        pltpu.async_copy(tab_sh.at[sidx.at[g + 2]],
                                         rows[bn], gs[bn])
                    else:
                        pltpu.make_async_copy(
                            rows[bn], acc_sh.at[didx.at[g - 2]],
                            ss[bn]).wait()

                        @pl.when(i < n_groups - 1)
                        def _():
                            pltpu.async_copy(tab_sh.at[sidx.at[g + 2]],
                                             rows[bn], gs[bn])
                return carry

            lax.fori_loop(0, n_groups, group, 0)
            pltpu.make_async_copy(r2, acc_sh.at[didx.at[cpp - 2]],
                                  s2).wait()
            pltpu.make_async_copy(r3, acc_sh.at[didx.at[cpp - 1]],
                                  s3).wait()
        plsc.subcore_barrier()
        pltpu.sync_copy(
            acc_sh.at[pl.ds(s * ROWS_PER_TILE, ROWS_PER_TILE)],
            out_hbm.at[c, pl.ds(s * ROWS_PER_TILE, ROWS_PER_TILE)])

    return scatter


def _make_deg_scatter(interpret=False):
    mesh = plsc.VectorSubcoreMesh(core_axis_name="c", subcore_axis_name="s",
                                  num_cores=NC, num_subcores=NS)

    @functools.partial(
        pl.kernel,
        out_type=jax.ShapeDtypeStruct((NC, NPAD, DEG_W), jnp.float32),
        mesh=mesh,
        scratch_types=[
            pltpu.VMEM_SHARED((NPAD, DEG_W), jnp.float32),
            pltpu.VMEM((CPT, CH), jnp.int32),
            pltpu.VMEM((CH, DEG_W), jnp.float32),
            pltpu.VMEM((CH, DEG_W), jnp.float32),
        ],
        compiler_params=pltpu.CompilerParams(use_tc_tiling_on_sc=False),
        interpret=interpret,
    )
    def deg(dst_hbm, out_hbm, acc_sh, didx, ones, zbuf):
        c = lax.axis_index("c")
        s = lax.axis_index("s")
        zeros16 = jnp.zeros((LANES,), jnp.float32)
        ones16 = jnp.ones((LANES,), jnp.float32)

        def fill(i, carry):
            zbuf[i, pl.ds(0, LANES)] = zeros16
            ones[i, pl.ds(0, LANES)] = ones16
            return carry

        lax.fori_loop(0, CH, fill, 0)
        for t in range(ROWS_PER_TILE // CH):
            pltpu.sync_copy(
                zbuf, acc_sh.at[pl.ds(s * ROWS_PER_TILE + t * CH, CH)])
        base = (c * NS + s) * CPT
        pltpu.sync_copy(dst_hbm.at[pl.ds(base, CPT)], didx)
        plsc.subcore_barrier()

        def chunk(k, carry):
            pltpu.sync_copy(ones, acc_sh.at[didx.at[k]], add=True)
            return carry

        lax.fori_loop(0, CPT, chunk, 0)
        plsc.subcore_barrier()
        pltpu.sync_copy(
            acc_sh.at[pl.ds(s * ROWS_PER_TILE, ROWS_PER_TILE)],
            out_hbm.at[c, pl.ds(s * ROWS_PER_TILE, ROWS_PER_TILE)])

    return deg


def _prep_body(pdeg_ref, x1_ref, x2_ref, w_ref, z_ref, dinv_ref):
    for g, x_ref in ((0, x1_ref), (1, x2_ref)):
        deg = pdeg_ref[g, :, 0:1] + 1.0
        dinv = lax.rsqrt(deg)
        dinv_ref[g * NPAD:(g + 1) * NPAD] = dinv
        h = jnp.dot(x_ref[...], w_ref[...],
                    preferred_element_type=jnp.float32)
        z_ref[g * NPAD:g * NPAD + N, :] = (dinv[0:N] * h).astype(
            jnp.bfloat16)
        z_ref[g * NPAD + N:(g + 1) * NPAD, :] = jnp.zeros(
            (NPAD - N, F1), jnp.bfloat16)


def _make_prep(interpret=False):
    return pl.pallas_call(
        _prep_body,
        out_shape=[
            jax.ShapeDtypeStruct((2 * NPAD, F1), jnp.bfloat16),
            jax.ShapeDtypeStruct((2 * NPAD, 1), jnp.float32),
        ],
        interpret=interpret,
    )


def _mid_body(p_ref, z_ref, dinv_ref, b_ref, w_ref, out_ref, *, F_out):
    a = (p_ref[...].astype(jnp.float32) + z_ref[...].astype(jnp.float32))
    u = jnp.maximum(dinv_ref[...] * a + b_ref[...], 0.0)
    h = jnp.dot(u, w_ref[...], preferred_element_type=jnp.float32)
    out_ref[...] = (dinv_ref[...] * h).astype(jnp.bfloat16)
    for g in range(2):
        out_ref[g * NPAD + N:(g + 1) * NPAD, :] = jnp.zeros(
            (NPAD - N, F_out), jnp.bfloat16)


def _make_mid(F_out, interpret=False):
    return pl.pallas_call(
        functools.partial(_mid_body, F_out=F_out),
        out_shape=jax.ShapeDtypeStruct((2 * NPAD, F_out), jnp.bfloat16),
        interpret=interpret,
    )


def _final_body(p_ref, z_ref, dinv_ref,
                b3_ref, watt_ref, wt_ref, v_ref, bt_ref,
                fc1_ref, fb1_ref, fc2_ref, fb2_ref, fc3_ref, fb3_ref,
                ws_ref, bs_ref, out_ref):
    def pooled(g):
        sl = pl.ds(g * NPAD, N)
        a = (p_ref[sl, :].astype(jnp.float32)
             + z_ref[sl, :].astype(jnp.float32))
        u = dinv_ref[sl, :] * a + b3_ref[...]
        m = jnp.dot(u, watt_ref[...], preferred_element_type=jnp.float32)
        ctx = jnp.tanh(jnp.mean(m, axis=0, keepdims=True))
        gate = jax.nn.sigmoid(lax.dot_general(
            u, ctx, (((1,), (1,)), ((), ())),
            preferred_element_type=jnp.float32))
        return lax.dot_general(
            u, gate, (((0,), (0,)), ((), ())),
            preferred_element_type=jnp.float32)

    h1 = pooled(0)
    h2 = pooled(1)

    sks = []
    for k in range(K):
        tk = jnp.dot(wt_ref[k], h2, preferred_element_type=jnp.float32)
        sks.append(lax.dot_general(
            h1, tk, (((0,), (0,)), ((), ())),
            preferred_element_type=jnp.float32))
    scoring = jnp.concatenate(sks, axis=1)
    comb = jnp.concatenate([h1, h2], axis=0)
    block = lax.dot_general(
        comb, v_ref[...], (((0,), (1,)), ((), ())),
        preferred_element_type=jnp.float32)
    srow = jnp.maximum(scoring + block + bt_ref[...], 0.0)
    srow = jnp.maximum(
        jnp.dot(srow, fc1_ref[...], preferred_element_type=jnp.float32)
        + fb1_ref[...], 0.0)
    srow = jnp.maximum(
        jnp.dot(srow, fc2_ref[...], preferred_element_type=jnp.float32)
        + fb2_ref[...], 0.0)
    srow = jnp.maximum(
        jnp.dot(srow, fc3_ref[...], preferred_element_type=jnp.float32)
        + fb3_ref[...], 0.0)
    out_ref[...] = jax.nn.sigmoid(
        jnp.dot(srow, ws_ref[...], preferred_element_type=jnp.float32)
        + bs_ref[...])


def _make_final(interpret=False):
    return pl.pallas_call(
        _final_body,
        out_shape=jax.ShapeDtypeStruct((1, 1), jnp.float32),
        interpret=interpret,
    )


@functools.lru_cache(maxsize=None)
def _kernels():
    return dict(
        deg=_make_deg_scatter(),
        scat128=_make_edge_scatter(F1, 64, 8),
        scat64=_make_edge_scatter(F2, 128, 1),
        scat32=_make_edge_scatter(F3, 128, 1),
        prep=_make_prep(),
        mid64=_make_mid(F2),
        mid32=_make_mid(F3),
        final=_make_final(),
    )


def _pad_edges(ei1, ei2):
    fill = jnp.full((EPAD - E,), N, dtype=jnp.int32)
    src = jnp.concatenate([ei1[0], fill, ei2[0], fill])
    dst = jnp.concatenate([ei1[1], fill, ei2[1], fill])
    return src, dst


def kernel(x1, edge_index_1, x2, edge_index_2, W1, b1, W2, b2, W3, b3,
           Watt, Wt, V, bt, FC1, fb1, FC2, fb2, FC3, fb3, WS, bs):
    ks = _kernels()
    src, dst = _pad_edges(edge_index_1, edge_index_2)
    src64, dst64 = src.reshape(-1, 64), dst.reshape(-1, 64)
    src128, dst128 = src.reshape(-1, CH), dst.reshape(-1, CH)
    pdeg = ks["deg"](dst128)
    z1, dinv = ks["prep"](pdeg, x1, x2, W1)
    p1 = ks["scat128"](z1, src64, dst64).reshape(2 * NPAD, F1)
    z2 = ks["mid64"](p1, z1, dinv, b1.reshape(1, F1), W2)
    p2 = ks["scat64"](z2, src128, dst128).reshape(2 * NPAD, F2)
    z3 = ks["mid32"](p2, z2, dinv, b2.reshape(1, F2), W3)
    p3 = ks["scat32"](z3, src128, dst128).reshape(2 * NPAD, F3)
    wt_r = jnp.transpose(Wt, (2, 0, 1))
    score = ks["final"](p3, z3, dinv,
                        b3.reshape(1, F3), Watt, wt_r, V, bt.reshape(1, K),
                        FC1, fb1.reshape(1, -1), FC2, fb2.reshape(1, -1),
                        FC3, fb3.reshape(1, -1), WS, bs.reshape(1, 1))
    return score.reshape(-1)


# speedup vs baseline: 2.7587x; 1.0361x over previous
import functools

import jax
import jax.numpy as jnp
from jax import lax
from jax.experimental import pallas as pl
from jax.experimental.pallas import tpu as pltpu
from jax.experimental.pallas import tpu_sc as plsc

N = 10000
D_IN = 128
F1, F2, F3 = 128, 64, 32
K = 16

NPAD = 10240
E = 320000
NC, NS, LANES = 2, 16, 16
CH = 128
CPT = 160
PHASES = 4
CPP = CPT // PHASES
EPAD = CH * CPT * NS
ROWS_PER_TILE = NPAD // NS
DEG_W = 16


def _make_edge_scatter(F, ch, phases, interpret=False):
    mesh = plsc.VectorSubcoreMesh(core_axis_name="c", subcore_axis_name="s",
                                  num_cores=NC, num_subcores=NS)
    cpt = (EPAD // NS) // ch
    cpp = cpt // phases
    n_groups = cpp // 4
    assert cpp % 4 == 0 and ROWS_PER_TILE % ch == 0

    @functools.partial(
        pl.kernel,
        out_type=jax.ShapeDtypeStruct((NC, NPAD, F), jnp.bfloat16),
        mesh=mesh,
        scratch_types=[
            pltpu.VMEM_SHARED((NPAD, F), jnp.bfloat16),
            pltpu.VMEM_SHARED((NPAD, F), jnp.bfloat16),
            pltpu.VMEM((cpp, ch), jnp.int32),
            pltpu.VMEM((cpp, ch), jnp.int32),
        ] + [pltpu.VMEM((ch, F), jnp.bfloat16)] * 4
          + [pltpu.SemaphoreType.DMA] * 8,
        compiler_params=pltpu.CompilerParams(use_tc_tiling_on_sc=False),
        interpret=interpret,
    )
    def scatter(z_hbm, src_hbm, dst_hbm, out_hbm, acc_sh, tab_sh, sidx, didx,
                r0, r1, r2, r3, g0, g1, g2, g3, s0, s1, s2, s3):
        c = lax.axis_index("c")
        s = lax.axis_index("s")
        rows = (r0, r1, r2, r3)
        gs = (g0, g1, g2, g3)
        ss = (s0, s1, s2, s3)
        zeros32 = jnp.zeros((2 * LANES,), jnp.bfloat16)

        def zrow(i, carry):
            for j in range(F // (2 * LANES)):
                r0[i, pl.ds(j * 2 * LANES, 2 * LANES)] = zeros32
            return carry

        lax.fori_loop(0, ch, zrow, 0)
        for t in range(ROWS_PER_TILE // ch):
            pltpu.sync_copy(
                r0, acc_sh.at[pl.ds(s * ROWS_PER_TILE + t * ch, ch)])
        pltpu.sync_copy(
            z_hbm.at[pl.ds(c * NPAD + s * ROWS_PER_TILE, ROWS_PER_TILE)],
            tab_sh.at[pl.ds(s * ROWS_PER_TILE, ROWS_PER_TILE)])
        plsc.subcore_barrier()

        base = (c * NS + s) * cpt
        for h in range(phases):
            pltpu.sync_copy(src_hbm.at[pl.ds(base + h * cpp, cpp)], sidx)
            pltpu.sync_copy(dst_hbm.at[pl.ds(base + h * cpp, cpp)], didx)
            pltpu.async_copy(tab_sh.at[sidx.at[0]], r0, g0)
            pltpu.async_copy(tab_sh.at[sidx.at[1]], r1, g1)

            def group(i, carry):
                for b in range(4):
                    g = 4 * i + b
                    bn = (b + 2) % 4
                    pltpu.make_async_copy(tab_sh.at[sidx.at[g]], rows[b],
                                          gs[b]).wait()
                    pltpu.async_copy(rows[b], acc_sh.at[didx.at[g]],
                                     ss[b], add=True)
                    if b < 2:
                        @pl.when(i > 0)
                        def _():
                            pltpu.make_async_copy(
                                rows[bn], acc_sh.at[didx.at[g - 2]],
                                ss[bn]).wait()
                        pltpu.async_copy(tab_sh.at[sidx.at[g + 2]],
                                         rows[bn], gs[bn])
                    else:
                        pltpu.make_async_copy(
                            rows[bn], acc_sh.at[didx.at[g - 2]],
                            ss[bn]).wait()

                        @pl.when(i < n_groups - 1)
                        def _():
                            pltpu.async_copy(tab_sh.at[sidx.at[g + 2]],
                                             rows[bn], gs[bn])
                return carry

            lax.fori_loop(0, n_groups, group, 0)
            pltpu.make_async_copy(r2, acc_sh.at[didx.at[cpp - 2]],
                                  s2).wait()
            pltpu.make_async_copy(r3, acc_sh.at[didx.at[cpp - 1]],
                                  s3).wait()
        plsc.subcore_barrier()
        pltpu.sync_copy(
            acc_sh.at[pl.ds(s * ROWS_PER_TILE, ROWS_PER_TILE)],
            out_hbm.at[c, pl.ds(s * ROWS_PER_TILE, ROWS_PER_TILE)])

    return scatter


def _make_deg_scatter(interpret=False):
    mesh = plsc.VectorSubcoreMesh(core_axis_name="c", subcore_axis_name="s",
                                  num_cores=NC, num_subcores=NS)

    @functools.partial(
        pl.kernel,
        out_type=jax.ShapeDtypeStruct((NC, NPAD, DEG_W), jnp.float32),
        mesh=mesh,
        scratch_types=[
            pltpu.VMEM_SHARED((NPAD, DEG_W), jnp.float32),
            pltpu.VMEM((CPT, CH), jnp.int32),
            pltpu.VMEM((CH, DEG_W), jnp.float32),
            pltpu.VMEM((CH, DEG_W), jnp.float32),
            pltpu.SemaphoreType.DMA,
            pltpu.SemaphoreType.DMA,
        ],
        compiler_params=pltpu.CompilerParams(use_tc_tiling_on_sc=False),
        interpret=interpret,
    )
    def deg(dst_hbm, out_hbm, acc_sh, didx, ones, zbuf, d0, d1):
        c = lax.axis_index("c")
        s = lax.axis_index("s")
        zeros16 = jnp.zeros((LANES,), jnp.float32)
        ones16 = jnp.ones((LANES,), jnp.float32)

        def fill(i, carry):
            zbuf[i, pl.ds(0, LANES)] = zeros16
            ones[i, pl.ds(0, LANES)] = ones16
            return carry

        lax.fori_loop(0, CH, fill, 0)
        for t in range(ROWS_PER_TILE // CH):
            pltpu.sync_copy(
                zbuf, acc_sh.at[pl.ds(s * ROWS_PER_TILE + t * CH, CH)])
        base = (c * NS + s) * CPT
        pltpu.sync_copy(dst_hbm.at[pl.ds(base, CPT)], didx)
        plsc.subcore_barrier()

        def chunk(k, carry):
            g = 2 * k
            pltpu.async_copy(ones, acc_sh.at[didx.at[g]], d0, add=True)
            pltpu.async_copy(ones, acc_sh.at[didx.at[g + 1]], d1, add=True)
            pltpu.make_async_copy(ones, acc_sh.at[didx.at[g]], d0).wait()
            pltpu.make_async_copy(ones, acc_sh.at[didx.at[g + 1]], d1).wait()
            return carry

        lax.fori_loop(0, CPT // 2, chunk, 0)
        plsc.subcore_barrier()
        pltpu.sync_copy(
            acc_sh.at[pl.ds(s * ROWS_PER_TILE, ROWS_PER_TILE)],
            out_hbm.at[c, pl.ds(s * ROWS_PER_TILE, ROWS_PER_TILE)])

    return deg


def _prep_body(pdeg_ref, x1_ref, x2_ref, w_ref, z_ref, dinv_ref):
    for g, x_ref in ((0, x1_ref), (1, x2_ref)):
        deg = pdeg_ref[g, :, 0:1] + 1.0
        dinv = lax.rsqrt(deg)
        dinv_ref[g * NPAD:(g + 1) * NPAD] = dinv
        h = jnp.dot(x_ref[...], w_ref[...],
                    preferred_element_type=jnp.float32)
        z_ref[g * NPAD:g * NPAD + N, :] = (dinv[0:N] * h).astype(
            jnp.bfloat16)
        z_ref[g * NPAD + N:(g + 1) * NPAD, :] = jnp.zeros(
            (NPAD - N, F1), jnp.bfloat16)


def _make_prep(interpret=False):
    return pl.pallas_call(
        _prep_body,
        out_shape=[
            jax.ShapeDtypeStruct((2 * NPAD, F1), jnp.bfloat16),
            jax.ShapeDtypeStruct((2 * NPAD, 1), jnp.float32),
        ],
        interpret=interpret,
    )


def _mid_body(p_ref, z_ref, dinv_ref, b_ref, w_ref, out_ref, *, F_out):
    a = (p_ref[...].astype(jnp.float32) + z_ref[...].astype(jnp.float32))
    u = jnp.maximum(dinv_ref[...] * a + b_ref[...], 0.0)
    h = jnp.dot(u, w_ref[...], preferred_element_type=jnp.float32)
    out_ref[...] = (dinv_ref[...] * h).astype(jnp.bfloat16)
    for g in range(2):
        out_ref[g * NPAD + N:(g + 1) * NPAD, :] = jnp.zeros(
            (NPAD - N, F_out), jnp.bfloat16)


def _make_mid(F_out, interpret=False):
    return pl.pallas_call(
        functools.partial(_mid_body, F_out=F_out),
        out_shape=jax.ShapeDtypeStruct((2 * NPAD, F_out), jnp.bfloat16),
        interpret=interpret,
    )


def _final_body(p_ref, z_ref, dinv_ref,
                b3_ref, watt_ref, wt_ref, v_ref, bt_ref,
                fc1_ref, fb1_ref, fc2_ref, fb2_ref, fc3_ref, fb3_ref,
                ws_ref, bs_ref, out_ref):
    def pooled(g):
        sl = pl.ds(g * NPAD, N)
        a = (p_ref[sl, :].astype(jnp.float32)
             + z_ref[sl, :].astype(jnp.float32))
        u = dinv_ref[sl, :] * a + b3_ref[...]
        m = jnp.dot(u, watt_ref[...], preferred_element_type=jnp.float32)
        ctx = jnp.tanh(jnp.mean(m, axis=0, keepdims=True))
        gate = jax.nn.sigmoid(lax.dot_general(
            u, ctx, (((1,), (1,)), ((), ())),
            preferred_element_type=jnp.float32))
        return lax.dot_general(
            u, gate, (((0,), (0,)), ((), ())),
            preferred_element_type=jnp.float32)

    h1 = pooled(0)
    h2 = pooled(1)

    sks = []
    for k in range(K):
        tk = jnp.dot(wt_ref[k], h2, preferred_element_type=jnp.float32)
        sks.append(lax.dot_general(
            h1, tk, (((0,), (0,)), ((), ())),
            preferred_element_type=jnp.float32))
    scoring = jnp.concatenate(sks, axis=1)
    comb = jnp.concatenate([h1, h2], axis=0)
    block = lax.dot_general(
        comb, v_ref[...], (((0,), (1,)), ((), ())),
        preferred_element_type=jnp.float32)
    srow = jnp.maximum(scoring + block + bt_ref[...], 0.0)
    srow = jnp.maximum(
        jnp.dot(srow, fc1_ref[...], preferred_element_type=jnp.float32)
        + fb1_ref[...], 0.0)
    srow = jnp.maximum(
        jnp.dot(srow, fc2_ref[...], preferred_element_type=jnp.float32)
        + fb2_ref[...], 0.0)
    srow = jnp.maximum(
        jnp.dot(srow, fc3_ref[...], preferred_element_type=jnp.float32)
        + fb3_ref[...], 0.0)
    out_ref[...] = jax.nn.sigmoid(
        jnp.dot(srow, ws_ref[...], preferred_element_type=jnp.float32)
        + bs_ref[...])


def _make_final(interpret=False):
    return pl.pallas_call(
        _final_body,
        out_shape=jax.ShapeDtypeStruct((1, 1), jnp.float32),
        interpret=interpret,
    )


@functools.lru_cache(maxsize=None)
def _kernels():
    return dict(
        deg=_make_deg_scatter(),
        scat128=_make_edge_scatter(F1, 128, 4),
        scat64=_make_edge_scatter(F2, 128, 1),
        scat32=_make_edge_scatter(F3, 128, 1),
        prep=_make_prep(),
        mid64=_make_mid(F2),
        mid32=_make_mid(F3),
        final=_make_final(),
    )


def _pad_edges(ei1, ei2):
    fill = jnp.full((EPAD - E,), N, dtype=jnp.int32)
    src = jnp.concatenate([ei1[0], fill, ei2[0], fill])
    dst = jnp.concatenate([ei1[1], fill, ei2[1], fill])
    return src, dst


def kernel(x1, edge_index_1, x2, edge_index_2, W1, b1, W2, b2, W3, b3,
           Watt, Wt, V, bt, FC1, fb1, FC2, fb2, FC3, fb3, WS, bs):
    ks = _kernels()
    src, dst = _pad_edges(edge_index_1, edge_index_2)
    src128, dst128 = src.reshape(-1, CH), dst.reshape(-1, CH)
    pdeg = ks["deg"](dst128)
    z1, dinv = ks["prep"](pdeg, x1, x2, W1)
    p1 = ks["scat128"](z1, src128, dst128).reshape(2 * NPAD, F1)
    z2 = ks["mid64"](p1, z1, dinv, b1.reshape(1, F1), W2)
    p2 = ks["scat64"](z2, src128, dst128).reshape(2 * NPAD, F2)
    z3 = ks["mid32"](p2, z2, dinv, b2.reshape(1, F2), W3)
    p3 = ks["scat32"](z3, src128, dst128).reshape(2 * NPAD, F3)
    wt_r = jnp.transpose(Wt, (2, 0, 1))
    score = ks["final"](p3, z3, dinv,
                        b3.reshape(1, F3), Watt, wt_r, V, bt.reshape(1, K),
                        FC1, fb1.reshape(1, -1), FC2, fb2.reshape(1, -1),
                        FC3, fb3.reshape(1, -1), WS, bs.reshape(1, 1))
    return score.reshape(-1)
